# baseline (speedup 1.0000x reference)
import functools

import jax
import jax.numpy as jnp
from jax import lax
from jax.experimental import pallas as pl
from jax.experimental.pallas import tpu as pltpu
from jax.experimental.pallas import tpu_sc as plsc

BATCH = 16384
EMBED = 64
NUM_CORES = 2
NUM_SUBCORES = 16
NUM_TILES = NUM_CORES * NUM_SUBCORES
B_PER_TILE = BATCH // NUM_TILES
IDX_CHUNK = 128
LANES = 16


def _sc_kernel(user_table, ad_table, user_ids, ad_ids, w_vec, b_vec):
    mesh = plsc.VectorSubcoreMesh(
        core_axis_name="c",
        subcore_axis_name="s",
        num_cores=NUM_CORES,
        num_subcores=NUM_SUBCORES,
    )

    cp = pltpu.CompilerParams(
        needs_layout_passes=False, use_tc_tiling_on_sc=False)

    @functools.partial(
        pl.kernel,
        out_type=jax.ShapeDtypeStruct((BATCH,), jnp.float32),
        mesh=mesh,
        compiler_params=cp,
        scratch_types=[
            pltpu.VMEM((B_PER_TILE,), jnp.int32),
            pltpu.VMEM((B_PER_TILE,), jnp.int32),
            pltpu.VMEM((B_PER_TILE, EMBED), jnp.float32),
            pltpu.VMEM((B_PER_TILE, EMBED), jnp.float32),
            pltpu.VMEM((B_PER_TILE,), jnp.float32),
            pltpu.VMEM((LANES,), jnp.float32),
            pltpu.VMEM((LANES,), jnp.float32),
            pltpu.SemaphoreType.DMA,
        ],
    )
    def kern(ut_hbm, at_hbm, uid_hbm, aid_hbm, w_hbm, b_hbm, out_hbm,
             uidx_v, aidx_v, urows_v, arows_v, out_v, w_v, b_v, sem):
        tile = lax.axis_index("s") * NUM_CORES + lax.axis_index("c")
        base = tile * B_PER_TILE

        pltpu.sync_copy(uid_hbm.at[pl.ds(base, B_PER_TILE)], uidx_v)
        pltpu.sync_copy(aid_hbm.at[pl.ds(base, B_PER_TILE)], aidx_v)
        pltpu.sync_copy(w_hbm, w_v)
        pltpu.sync_copy(b_hbm, b_v)

        n_chunks = B_PER_TILE // IDX_CHUNK
        copies = []
        for j in range(n_chunks):
            sl = pl.ds(j * IDX_CHUNK, IDX_CHUNK)
            copies.append(pltpu.async_copy(
                ut_hbm.at[uidx_v.at[sl]], urows_v.at[sl], sem))
            copies.append(pltpu.async_copy(
                at_hbm.at[aidx_v.at[sl]], arows_v.at[sl], sem))
        for c in copies:
            c.wait()

        w = w_v[...]
        b = b_v[...]

        @pl.loop(0, B_PER_TILE, step=LANES)
        def _(g):
            row_ids = lax.iota(jnp.int32, LANES) + g
            acc = jnp.zeros((LANES,), jnp.float32)
            for j in range(EMBED):
                col = jnp.full((LANES,), j, jnp.int32)
                u = plsc.load_gather(urows_v, [row_ids, col])
                a = plsc.load_gather(arows_v, [row_ids, col])
                acc = acc + u * a
            z = acc * w + b
            out_v[pl.ds(g, LANES)] = 1.0 / (1.0 + jnp.exp(-z))

        pltpu.sync_copy(out_v, out_hbm.at[pl.ds(base, B_PER_TILE)])

    return kern(user_table, ad_table, user_ids, ad_ids, w_vec, b_vec)


@jax.jit
def kernel(user_ids, ad_ids, user_table, ad_table, fc_w, fc_b):
    w_vec = jnp.broadcast_to(fc_w.reshape(()), (LANES,)).astype(jnp.float32)
    b_vec = jnp.broadcast_to(fc_b.reshape(()), (LANES,)).astype(jnp.float32)
    out = _sc_kernel(user_table, ad_table, user_ids.astype(jnp.int32),
                     ad_ids.astype(jnp.int32), w_vec, b_vec)
    return out.reshape(BATCH, 1)


# speedup vs baseline: 1.6993x; 1.6993x over previous
import functools

import jax
import jax.numpy as jnp
from jax import lax
from jax.experimental import pallas as pl
from jax.experimental.pallas import tpu as pltpu
from jax.experimental.pallas import tpu_sc as plsc

BATCH = 16384
EMBED = 64
NUM_CORES = 2
NUM_SUBCORES = 16
NUM_TILES = NUM_CORES * NUM_SUBCORES
B_PER_TILE = BATCH // NUM_TILES
PASS_ROWS = 32
N_PASS = B_PER_TILE // PASS_ROWS
LANES = 16


def _sc_kernel(user_table, ad_table, user_ids, ad_ids, w_vec, b_vec):
    mesh = plsc.VectorSubcoreMesh(
        core_axis_name="c",
        subcore_axis_name="s",
        num_cores=NUM_CORES,
        num_subcores=NUM_SUBCORES,
    )

    cp = pltpu.CompilerParams(
        needs_layout_passes=False, disable_bounds_checks=True)

    @functools.partial(
        pl.kernel,
        out_type=jax.ShapeDtypeStruct((BATCH,), jnp.float32),
        mesh=mesh,
        compiler_params=cp,
        scratch_types=[
            pltpu.VMEM((B_PER_TILE,), jnp.int32),
            pltpu.VMEM((B_PER_TILE,), jnp.int32),
            pltpu.VMEM((2, PASS_ROWS, EMBED), jnp.float32),
            pltpu.VMEM((2, PASS_ROWS, EMBED), jnp.float32),
            pltpu.VMEM((B_PER_TILE,), jnp.float32),
            pltpu.VMEM((LANES,), jnp.float32),
            pltpu.VMEM((LANES,), jnp.float32),
            pltpu.SemaphoreType.DMA,
            pltpu.SemaphoreType.DMA,
        ],
    )
    def kern(ut_hbm, at_hbm, uid_hbm, aid_hbm, w_hbm, b_hbm, out_hbm,
             uidx_v, aidx_v, ubuf_v, abuf_v, out_v, w_v, b_v, sem0, sem1):
        tile = lax.axis_index("s") * NUM_CORES + lax.axis_index("c")
        base = tile * B_PER_TILE

        pltpu.sync_copy(uid_hbm.at[pl.ds(base, B_PER_TILE)], uidx_v)
        pltpu.sync_copy(aid_hbm.at[pl.ds(base, B_PER_TILE)], aidx_v)
        pltpu.sync_copy(w_hbm, w_v)
        pltpu.sync_copy(b_hbm, b_v)

        sems = [sem0, sem1]

        def fire(h, buf):
            sem = sems[buf]
            for c in range(PASS_ROWS // LANES):
                uv = uidx_v[pl.ds(h * PASS_ROWS + c * LANES, LANES)]
                av = aidx_v[pl.ds(h * PASS_ROWS + c * LANES, LANES)]
                for l in range(LANES):
                    r = c * LANES + l
                    pltpu.async_copy(
                        ut_hbm.at[pl.ds(uv[l], 1), :],
                        ubuf_v.at[buf, pl.ds(r, 1), :], sem)
                    pltpu.async_copy(
                        at_hbm.at[pl.ds(av[l], 1), :],
                        abuf_v.at[buf, pl.ds(r, 1), :], sem)

        def drain(buf):
            pltpu.make_async_copy(
                ut_hbm.at[pl.ds(0, PASS_ROWS), :],
                ubuf_v.at[buf], sems[buf]).wait()
            pltpu.make_async_copy(
                at_hbm.at[pl.ds(0, PASS_ROWS), :],
                abuf_v.at[buf], sems[buf]).wait()

        w = w_v[...]
        b = b_v[...]
        lane = lax.iota(jnp.int32, LANES)

        def compute(h, buf):
            for grp in range(PASS_ROWS // LANES):
                off = h * PASS_ROWS + grp * LANES
                rows = lane + grp * LANES
                acc = jnp.zeros((LANES,), jnp.float32)
                for j in range(EMBED):
                    cols = (lane + j) & (EMBED - 1)
                    u = plsc.load_gather(ubuf_v.at[buf], [rows, cols])
                    a = plsc.load_gather(abuf_v.at[buf], [rows, cols])
                    acc = acc + u * a
                z = acc * w + b
                out_v[pl.ds(off, LANES)] = 1.0 / (1.0 + jnp.exp(-z))

        fire(0, 0)

        @pl.loop(0, N_PASS // 2)
        def _(i):
            p0 = 2 * i
            fire(p0 + 1, 1)
            drain(0)
            compute(p0, 0)

            @pl.when(p0 + 2 < N_PASS)
            def _():
                fire(p0 + 2, 0)

            drain(1)
            compute(p0 + 1, 1)

        pltpu.sync_copy(out_v, out_hbm.at[pl.ds(base, B_PER_TILE)])

    return kern(user_table, ad_table, user_ids, ad_ids, w_vec, b_vec)


@jax.jit
def kernel(user_ids, ad_ids, user_table, ad_table, fc_w, fc_b):
    w_vec = jnp.broadcast_to(fc_w.reshape(()), (LANES,)).astype(jnp.float32)
    b_vec = jnp.broadcast_to(fc_b.reshape(()), (LANES,)).astype(jnp.float32)
    out = _sc_kernel(user_table, ad_table, user_ids.astype(jnp.int32),
                     ad_ids.astype(jnp.int32), w_vec, b_vec)
    return out.reshape(BATCH, 1)
